# baseline (speedup 1.0000x reference)
import dataclasses

import jax
import jax.numpy as jnp
from jax.experimental import pallas as pl
from jax.experimental.pallas import tpu as pltpu
from jax.experimental.pallas import tpu_sc as plsc

_N = 10000
_E = 160000
_D = 256
_H = 128
_NP = 10240
_CH = 128
_NCHUNK = _E // _CH
_NSUB = 16
_RPS = _NP // _NSUB

_mesh = plsc.VectorSubcoreMesh(core_axis_name="c", subcore_axis_name="s")

_sc_params = pltpu.CompilerParams()
if "needs_layout_passes" in pltpu.CompilerParams.__dataclass_fields__:
    _sc_params = dataclasses.replace(_sc_params, needs_layout_passes=False)


def _deg_body(de_hbm, degp_hbm, didx, ones_v, zb, deg_small, acc_deg):
    c = jax.lax.axis_index("c")
    s = jax.lax.axis_index("s")
    ones16 = jnp.ones((16,), jnp.float32)
    zeros16 = jnp.zeros((16,), jnp.float32)

    @pl.loop(0, _CH)
    def _(i):
        ro = ones_v.at[i]
        rz = zb.at[i]
        for k in range(8):
            ro[pl.ds(k * 16, 16)] = ones16
            rz[pl.ds(k * 16, 16)] = zeros16

    base = s * _RPS

    @pl.loop(0, 5)
    def _(k):
        pltpu.sync_copy(zb, acc_deg.at[pl.ds(base + k * _CH, _CH)])

    plsc.subcore_barrier()

    @pl.loop(c * 625 + s, (c + 1) * 625, step=_NSUB)
    def _(j):
        pltpu.sync_copy(de_hbm.at[pl.ds(j * _CH, _CH)], didx)
        pltpu.sync_copy(ones_v, acc_deg.at[didx], add=True)

    plsc.subcore_barrier()

    iota16 = jax.lax.iota(jnp.int32, 16)
    zidx16 = jnp.zeros((16,), jnp.int32)

    @pl.loop(0, 5)
    def _(k):
        pltpu.sync_copy(acc_deg.at[pl.ds(base + k * _CH, _CH)], zb)

        @pl.loop(0, 8)
        def _(g):
            deg_small[pl.ds(k * _CH + g * 16, 16)] = plsc.load_gather(
                zb, [iota16 + g * 16, zidx16])

    pltpu.sync_copy(deg_small, degp_hbm.at[c].at[pl.ds(base, _RPS)])


_deg_call = pl.kernel(
    _deg_body,
    out_type=jax.ShapeDtypeStruct((2, _NP), jnp.float32),
    mesh=_mesh,
    compiler_params=_sc_params,
    scratch_types=[
        pltpu.VMEM((_CH,), jnp.int32),
        pltpu.VMEM((_CH, _H), jnp.float32),
        pltpu.VMEM((_CH, _H), jnp.float32),
        pltpu.VMEM((_RPS,), jnp.float32),
        pltpu.VMEM_SHARED((_NP, _H), jnp.float32),
    ],
)


def _dis_kernel(degp_ref, dis_ref):
    deg = degp_ref[0] + degp_ref[1]
    dis_ref[...] = jnp.where(deg > 0, jax.lax.rsqrt(jnp.maximum(deg, 1.0)), 0.0)


_dis_call = pl.pallas_call(
    _dis_kernel,
    out_shape=jax.ShapeDtypeStruct((_NP,), jnp.float32),
)


def _mm_kernel(x_ref, w_ref, b_ref, h_ref):
    h_ref[...] = jnp.dot(x_ref[...], w_ref[...],
                         preferred_element_type=jnp.float32) + b_ref[...]


_BR = 1024

_mm_call = pl.pallas_call(
    _mm_kernel,
    grid=(_NP // _BR,),
    in_specs=[
        pl.BlockSpec((_BR, _D), lambda q: (q, 0)),
        pl.BlockSpec((_D, _D), lambda q: (0, 0)),
        pl.BlockSpec((1, _D), lambda q: (0, 0)),
    ],
    out_specs=pl.BlockSpec((_BR, _D), lambda q: (q, 0)),
    out_shape=jax.ShapeDtypeStruct((_NP, _D), jnp.float32),
)


def _scale_kernel(h_ref, dis_ref, hl_ref, hr_ref):
    q = pl.program_id(0)
    hs = h_ref[...] * dis_ref[pl.ds(q * _BR, _BR)][:, None]
    hl_ref[...] = hs[:, :_H]
    hr_ref[...] = hs[:, _H:]


_scale_call = pl.pallas_call(
    _scale_kernel,
    grid=(_NP // _BR,),
    in_specs=[
        pl.BlockSpec((_BR, _D), lambda q: (q, 0)),
        pl.BlockSpec((_NP,), lambda q: (0,)),
    ],
    out_specs=[
        pl.BlockSpec((_BR, _H), lambda q: (q, 0)),
        pl.BlockSpec((_BR, _H), lambda q: (q, 0)),
    ],
    out_shape=[
        jax.ShapeDtypeStruct((_NP, _H), jnp.float32),
        jax.ShapeDtypeStruct((_NP, _H), jnp.float32),
    ],
)


def _main_body(hsl, hsr, dis_hbm, se, de, sl, dl, out_hbm,
               sidx, didx, sidx2, didx2, rows_v, zd_v, dis_own,
               dots, lg_v, semA, semB, acc):
    c = jax.lax.axis_index("c")
    s = jax.lax.axis_index("s")
    zeros16 = jnp.zeros((16,), jnp.float32)

    @pl.loop(0, _CH)
    def _(i):
        row = rows_v.at[i]
        for k in range(8):
            row[pl.ds(k * 16, 16)] = zeros16

    base = s * _RPS

    @pl.loop(0, 5)
    def _(k):
        pltpu.sync_copy(rows_v, acc.at[pl.ds(base + k * _CH, _CH)])

    pltpu.sync_copy(dis_hbm.at[pl.ds(base, _RPS)], dis_own)
    plsc.subcore_barrier()

    def _gather_hs(idx_ref, dst_ref, sem):
        @pl.when(c == 0)
        def _():
            pltpu.async_copy(hsl.at[idx_ref], dst_ref, sem).wait()

        @pl.when(c == 1)
        def _():
            pltpu.async_copy(hsr.at[idx_ref], dst_ref, sem).wait()

    def _start_gather_hs(idx_ref, dst_ref, sem, body):
        @pl.when(c == 0)
        def _():
            d = pltpu.async_copy(hsl.at[idx_ref], dst_ref, sem)
            body()
            d.wait()

        @pl.when(c == 1)
        def _():
            d = pltpu.async_copy(hsr.at[idx_ref], dst_ref, sem)
            body()
            d.wait()

    @pl.loop(0, 39)
    def _(t):
        jA = s + 32 * t
        jB = jA + 16
        pltpu.sync_copy(se.at[pl.ds(jA * _CH, _CH)], sidx)
        pltpu.sync_copy(de.at[pl.ds(jA * _CH, _CH)], didx)

        def _prefetch_idx_b():
            pltpu.sync_copy(se.at[pl.ds(jB * _CH, _CH)], sidx2)
            pltpu.sync_copy(de.at[pl.ds(jB * _CH, _CH)], didx2)

        _start_gather_hs(sidx, rows_v, semA, _prefetch_idx_b)

        def _scatter_a():
            pltpu.sync_copy(rows_v, acc.at[didx], add=True)

        _start_gather_hs(sidx2, zd_v, semB, _scatter_a)
        pltpu.sync_copy(zd_v, acc.at[didx2], add=True)

    jT = s + 32 * 39
    @pl.when(jT < _NCHUNK)
    def _():
        pltpu.sync_copy(se.at[pl.ds(jT * _CH, _CH)], sidx)
        pltpu.sync_copy(de.at[pl.ds(jT * _CH, _CH)], didx)
        _gather_hs(sidx, rows_v, semA)
        pltpu.sync_copy(rows_v, acc.at[didx], add=True)

    plsc.subcore_barrier()

    zidx16 = jnp.zeros((16,), jnp.int32)

    @pl.loop(0, 5)
    def _(k):
        pltpu.sync_copy(acc.at[pl.ds(base + k * _CH, _CH)], rows_v)

        @pl.loop(0, _CH)
        def _(r):
            d16 = plsc.load_gather(dis_own, [zidx16 + (k * _CH + r)])
            row = rows_v.at[r]
            for kk in range(8):
                row[pl.ds(kk * 16, 16)] = row[pl.ds(kk * 16, 16)] * d16

        pltpu.sync_copy(rows_v, acc.at[pl.ds(base + k * _CH, _CH)])

    plsc.subcore_barrier()

    iota16 = jax.lax.iota(jnp.int32, 16)
    i16x = iota16 * 16

    def _dot_half(zbuf, eoff):
        @plsc.parallel_loop(0, 4)
        def _(g):
            gb = g * 16
            db = g * 256
            for e in range(16):
                zr = zbuf.at[gb + e]
                dr = zbuf.at[64 + gb + e]
                a = zr[pl.ds(0, 16)] * dr[pl.ds(0, 16)]
                for k in range(1, 8):
                    a = a + zr[pl.ds(k * 16, 16)] * dr[pl.ds(k * 16, 16)]
                plsc.store_scatter(dots, [db + i16x + e], a)
            red = dots[pl.ds(db, 16)]
            for r in range(1, 16):
                red = red + dots[pl.ds(db + r * 16, 16)]
            lg_v[pl.ds(eoff + gb, 16)] = red

    @pl.loop(s, _NCHUNK, step=_NSUB)
    def _(j):
        pltpu.sync_copy(sl.at[pl.ds(j * _CH, _CH)], sidx)
        pltpu.sync_copy(dl.at[pl.ds(j * _CH, _CH)], didx)
        a1 = pltpu.async_copy(acc.at[sidx.at[pl.ds(0, 64)]],
                              rows_v.at[pl.ds(0, 64)], semA)
        a2 = pltpu.async_copy(acc.at[didx.at[pl.ds(0, 64)]],
                              rows_v.at[pl.ds(64, 64)], semA)
        b1 = pltpu.async_copy(acc.at[sidx.at[pl.ds(64, 64)]],
                              zd_v.at[pl.ds(0, 64)], semB)
        b2 = pltpu.async_copy(acc.at[didx.at[pl.ds(64, 64)]],
                              zd_v.at[pl.ds(64, 64)], semB)
        a1.wait()
        a2.wait()
        _dot_half(rows_v, 0)
        b1.wait()
        b2.wait()
        _dot_half(zd_v, 64)
        pltpu.sync_copy(lg_v, out_hbm.at[c].at[pl.ds(j * _CH, _CH)])


_main_call = pl.kernel(
    _main_body,
    out_type=jax.ShapeDtypeStruct((2, _E), jnp.float32),
    mesh=_mesh,
    compiler_params=_sc_params,
    scratch_types=[
        pltpu.VMEM((_CH,), jnp.int32),
        pltpu.VMEM((_CH,), jnp.int32),
        pltpu.VMEM((_CH,), jnp.int32),
        pltpu.VMEM((_CH,), jnp.int32),
        pltpu.VMEM((_CH, _H), jnp.float32),
        pltpu.VMEM((_CH, _H), jnp.float32),
        pltpu.VMEM((_RPS,), jnp.float32),
        pltpu.VMEM((1024,), jnp.float32),
        pltpu.VMEM((_CH,), jnp.float32),
        pltpu.SemaphoreType.DMA,
        pltpu.SemaphoreType.DMA,
        pltpu.VMEM_SHARED((_NP, _H), jnp.float32),
    ],
)


def _add_kernel(p_ref, o_ref):
    o_ref[...] = p_ref[0] + p_ref[1]


_add_call = pl.pallas_call(
    _add_kernel,
    out_shape=jax.ShapeDtypeStruct((_E,), jnp.float32),
)


def kernel(x, edge_index, edge_label_index, W, b):
    se = edge_index[0]
    de = edge_index[1]
    sl = edge_label_index[0]
    dl = edge_label_index[1]

    xp = jnp.pad(x, ((0, _NP - _N), (0, 0)))
    h = _mm_call(xp, W, b.reshape(1, _D))
    degp = _deg_call(de)
    dis = _dis_call(degp)
    hsl, hsr = _scale_call(h, dis)
    partial = _main_call(hsl, hsr, dis, se, de, sl, dl)
    return _add_call(partial)


# speedup vs baseline: 1.0944x; 1.0944x over previous
import dataclasses

import jax
import jax.numpy as jnp
from jax.experimental import pallas as pl
from jax.experimental.pallas import tpu as pltpu
from jax.experimental.pallas import tpu_sc as plsc

_N = 10000
_E = 160000
_D = 256
_H = 128
_NP = 10240
_CH = 128
_NCHUNK = _E // _CH
_NSUB = 16
_RPS = _NP // _NSUB

_mesh = plsc.VectorSubcoreMesh(core_axis_name="c", subcore_axis_name="s")

_sc_params = pltpu.CompilerParams()
if "needs_layout_passes" in pltpu.CompilerParams.__dataclass_fields__:
    _sc_params = dataclasses.replace(_sc_params, needs_layout_passes=False)


def _deg_body(de_hbm, degp_hbm, didx, didx2, ones_v, zb, deg_small, semK, semK2,
              acc_deg):
    c = jax.lax.axis_index("c")
    s = jax.lax.axis_index("s")
    ones16 = jnp.ones((16,), jnp.float32)
    zeros16 = jnp.zeros((16,), jnp.float32)

    @pl.loop(0, _CH)
    def _(i):
        ro = ones_v.at[i]
        rz = zb.at[i]
        for k in range(8):
            ro[pl.ds(k * 16, 16)] = ones16
            rz[pl.ds(k * 16, 16)] = zeros16

    base = s * _RPS

    @pl.loop(0, 5)
    def _(k):
        pltpu.sync_copy(zb, acc_deg.at[pl.ds(base + k * _CH, _CH)])

    plsc.subcore_barrier()

    base0 = c * 625 + s

    @pl.loop(0, 19)
    def _(t):
        jA = base0 + 32 * t
        jB = jA + 16
        pltpu.sync_copy(de_hbm.at[pl.ds(jA * _CH, _CH)], didx)
        dA = pltpu.async_copy(ones_v, acc_deg.at[didx], semK, add=True)
        pltpu.sync_copy(de_hbm.at[pl.ds(jB * _CH, _CH)], didx2)
        dB = pltpu.async_copy(ones_v, acc_deg.at[didx2], semK2, add=True)
        dA.wait()
        dB.wait()

    j38 = base0 + 16 * 38
    pltpu.sync_copy(de_hbm.at[pl.ds(j38 * _CH, _CH)], didx)
    pltpu.sync_copy(ones_v, acc_deg.at[didx], add=True)

    @pl.when(s == 0)
    def _():
        j39 = base0 + 16 * 39
        pltpu.sync_copy(de_hbm.at[pl.ds(j39 * _CH, _CH)], didx2)
        pltpu.sync_copy(ones_v, acc_deg.at[didx2], add=True)

    plsc.subcore_barrier()

    iota16 = jax.lax.iota(jnp.int32, 16)
    zidx16 = jnp.zeros((16,), jnp.int32)

    @pl.loop(0, 5)
    def _(k):
        pltpu.sync_copy(acc_deg.at[pl.ds(base + k * _CH, _CH)], zb)

        @pl.loop(0, 8)
        def _(g):
            deg_small[pl.ds(k * _CH + g * 16, 16)] = plsc.load_gather(
                zb, [iota16 + g * 16, zidx16])

    pltpu.sync_copy(deg_small, degp_hbm.at[c].at[pl.ds(base, _RPS)])


_deg_call = pl.kernel(
    _deg_body,
    out_type=jax.ShapeDtypeStruct((2, _NP), jnp.float32),
    mesh=_mesh,
    compiler_params=_sc_params,
    scratch_types=[
        pltpu.VMEM((_CH,), jnp.int32),
        pltpu.VMEM((_CH,), jnp.int32),
        pltpu.VMEM((_CH, _H), jnp.float32),
        pltpu.VMEM((_CH, _H), jnp.float32),
        pltpu.VMEM((_RPS,), jnp.float32),
        pltpu.SemaphoreType.DMA,
        pltpu.SemaphoreType.DMA,
        pltpu.VMEM_SHARED((_NP, _H), jnp.float32),
    ],
)


def _dis_kernel(degp_ref, dis_ref):
    deg = degp_ref[0] + degp_ref[1]
    dis_ref[...] = jnp.where(deg > 0, jax.lax.rsqrt(jnp.maximum(deg, 1.0)), 0.0)


_dis_call = pl.pallas_call(
    _dis_kernel,
    out_shape=jax.ShapeDtypeStruct((_NP,), jnp.float32),
)


def _mm_kernel(x_ref, w_ref, b_ref, h_ref):
    h_ref[...] = jnp.dot(x_ref[...], w_ref[...],
                         preferred_element_type=jnp.float32) + b_ref[...]


_BR = 1024

_mm_call = pl.pallas_call(
    _mm_kernel,
    grid=(_NP // _BR,),
    in_specs=[
        pl.BlockSpec((_BR, _D), lambda q: (q, 0)),
        pl.BlockSpec((_D, _D), lambda q: (0, 0)),
        pl.BlockSpec((1, _D), lambda q: (0, 0)),
    ],
    out_specs=pl.BlockSpec((_BR, _D), lambda q: (q, 0)),
    out_shape=jax.ShapeDtypeStruct((_NP, _D), jnp.float32),
)


def _scale_kernel(h_ref, dis_ref, hl_ref, hr_ref):
    q = pl.program_id(0)
    hs = h_ref[...] * dis_ref[pl.ds(q * _BR, _BR)][:, None]
    hl_ref[...] = hs[:, :_H]
    hr_ref[...] = hs[:, _H:]


_scale_call = pl.pallas_call(
    _scale_kernel,
    grid=(_NP // _BR,),
    in_specs=[
        pl.BlockSpec((_BR, _D), lambda q: (q, 0)),
        pl.BlockSpec((_NP,), lambda q: (0,)),
    ],
    out_specs=[
        pl.BlockSpec((_BR, _H), lambda q: (q, 0)),
        pl.BlockSpec((_BR, _H), lambda q: (q, 0)),
    ],
    out_shape=[
        jax.ShapeDtypeStruct((_NP, _H), jnp.float32),
        jax.ShapeDtypeStruct((_NP, _H), jnp.float32),
    ],
)


def _main_body(hsl, hsr, dis_hbm, se, de, sl, dl, out_hbm,
               sidx, didx, sidx2, didx2, rows_v, zd_v, dis_own,
               dots, lg_v, semA, semB, acc):
    c = jax.lax.axis_index("c")
    s = jax.lax.axis_index("s")
    zeros16 = jnp.zeros((16,), jnp.float32)

    @pl.loop(0, _CH)
    def _(i):
        row = rows_v.at[i]
        for k in range(8):
            row[pl.ds(k * 16, 16)] = zeros16

    base = s * _RPS

    @pl.loop(0, 5)
    def _(k):
        pltpu.sync_copy(rows_v, acc.at[pl.ds(base + k * _CH, _CH)])

    pltpu.sync_copy(dis_hbm.at[pl.ds(base, _RPS)], dis_own)
    plsc.subcore_barrier()

    def _gather_hs(idx_ref, dst_ref, sem):
        @pl.when(c == 0)
        def _():
            pltpu.async_copy(hsl.at[idx_ref], dst_ref, sem).wait()

        @pl.when(c == 1)
        def _():
            pltpu.async_copy(hsr.at[idx_ref], dst_ref, sem).wait()

    def _start_gather_hs(idx_ref, dst_ref, sem, body):
        @pl.when(c == 0)
        def _():
            d = pltpu.async_copy(hsl.at[idx_ref], dst_ref, sem)
            body()
            d.wait()

        @pl.when(c == 1)
        def _():
            d = pltpu.async_copy(hsr.at[idx_ref], dst_ref, sem)
            body()
            d.wait()

    @pl.loop(0, 39)
    def _(t):
        jA = s + 32 * t
        jB = jA + 16
        pltpu.sync_copy(se.at[pl.ds(jA * _CH, _CH)], sidx)
        pltpu.sync_copy(de.at[pl.ds(jA * _CH, _CH)], didx)

        def _prefetch_idx_b():
            pltpu.sync_copy(se.at[pl.ds(jB * _CH, _CH)], sidx2)
            pltpu.sync_copy(de.at[pl.ds(jB * _CH, _CH)], didx2)

        _start_gather_hs(sidx, rows_v, semA, _prefetch_idx_b)

        def _scatter_a():
            pltpu.sync_copy(rows_v, acc.at[didx], add=True)

        _start_gather_hs(sidx2, zd_v, semB, _scatter_a)
        pltpu.sync_copy(zd_v, acc.at[didx2], add=True)

    jT = s + 32 * 39
    @pl.when(jT < _NCHUNK)
    def _():
        pltpu.sync_copy(se.at[pl.ds(jT * _CH, _CH)], sidx)
        pltpu.sync_copy(de.at[pl.ds(jT * _CH, _CH)], didx)
        _gather_hs(sidx, rows_v, semA)
        pltpu.sync_copy(rows_v, acc.at[didx], add=True)

    plsc.subcore_barrier()

    zidx16 = jnp.zeros((16,), jnp.int32)

    @pl.loop(0, 5)
    def _(k):
        pltpu.sync_copy(acc.at[pl.ds(base + k * _CH, _CH)], rows_v)

        @pl.loop(0, _CH)
        def _(r):
            d16 = plsc.load_gather(dis_own, [zidx16 + (k * _CH + r)])
            row = rows_v.at[r]
            for kk in range(8):
                row[pl.ds(kk * 16, 16)] = row[pl.ds(kk * 16, 16)] * d16

        pltpu.sync_copy(rows_v, acc.at[pl.ds(base + k * _CH, _CH)])

    plsc.subcore_barrier()

    iota16 = jax.lax.iota(jnp.int32, 16)
    i16x = iota16 * 16

    def _dot_half(zbuf, eoff):
        @pl.loop(0, 4)
        def _(g):
            gb = g * 16
            for e in range(16):
                zr = zbuf.at[gb + e]
                dr = zbuf.at[64 + gb + e]
                a = zr[pl.ds(0, 16)] * dr[pl.ds(0, 16)]
                for k in range(1, 8):
                    a = a + zr[pl.ds(k * 16, 16)] * dr[pl.ds(k * 16, 16)]
                plsc.store_scatter(dots, [i16x + e], a)
            red = dots[pl.ds(0, 16)]
            for r in range(1, 16):
                red = red + dots[pl.ds(r * 16, 16)]
            lg_v[pl.ds(eoff + gb, 16)] = red

    @pl.loop(s, _NCHUNK, step=_NSUB)
    def _(j):
        pltpu.sync_copy(sl.at[pl.ds(j * _CH, _CH)], sidx)
        pltpu.sync_copy(dl.at[pl.ds(j * _CH, _CH)], didx)
        a1 = pltpu.async_copy(acc.at[sidx.at[pl.ds(0, 64)]],
                              rows_v.at[pl.ds(0, 64)], semA)
        a2 = pltpu.async_copy(acc.at[didx.at[pl.ds(0, 64)]],
                              rows_v.at[pl.ds(64, 64)], semA)
        b1 = pltpu.async_copy(acc.at[sidx.at[pl.ds(64, 64)]],
                              zd_v.at[pl.ds(0, 64)], semB)
        b2 = pltpu.async_copy(acc.at[didx.at[pl.ds(64, 64)]],
                              zd_v.at[pl.ds(64, 64)], semB)
        a1.wait()
        a2.wait()
        _dot_half(rows_v, 0)
        b1.wait()
        b2.wait()
        _dot_half(zd_v, 64)
        pltpu.sync_copy(lg_v, out_hbm.at[c].at[pl.ds(j * _CH, _CH)])


_main_call = pl.kernel(
    _main_body,
    out_type=jax.ShapeDtypeStruct((2, _E), jnp.float32),
    mesh=_mesh,
    compiler_params=_sc_params,
    scratch_types=[
        pltpu.VMEM((_CH,), jnp.int32),
        pltpu.VMEM((_CH,), jnp.int32),
        pltpu.VMEM((_CH,), jnp.int32),
        pltpu.VMEM((_CH,), jnp.int32),
        pltpu.VMEM((_CH, _H), jnp.float32),
        pltpu.VMEM((_CH, _H), jnp.float32),
        pltpu.VMEM((_RPS,), jnp.float32),
        pltpu.VMEM((256,), jnp.float32),
        pltpu.VMEM((_CH,), jnp.float32),
        pltpu.SemaphoreType.DMA,
        pltpu.SemaphoreType.DMA,
        pltpu.VMEM_SHARED((_NP, _H), jnp.float32),
    ],
)


def _add_kernel(p_ref, o_ref):
    o_ref[...] = p_ref[0] + p_ref[1]


_add_call = pl.pallas_call(
    _add_kernel,
    out_shape=jax.ShapeDtypeStruct((_E,), jnp.float32),
)


def kernel(x, edge_index, edge_label_index, W, b):
    se = edge_index[0]
    de = edge_index[1]
    sl = edge_label_index[0]
    dl = edge_label_index[1]

    xp = jnp.pad(x, ((0, _NP - _N), (0, 0)))
    h = _mm_call(xp, W, b.reshape(1, _D))
    degp = _deg_call(de)
    dis = _dis_call(degp)
    hsl, hsr = _scale_call(h, dis)
    partial = _main_call(hsl, hsr, dis, se, de, sl, dl)
    return _add_call(partial)
